# Initial kernel scaffold; baseline (speedup 1.0000x reference)
import jax
import jax.numpy as jnp
from jax.experimental import pallas as pl


def kernel(x, edge_index, W1, b1, W2, b2, W3, b3, Wl, bl):
    raise NotImplementedError("write your pallas kernel here")


# speedup vs baseline: 31.9638x; 31.9638x over previous
import functools

import jax
import jax.numpy as jnp
from jax import lax
from jax.experimental import pallas as pl
from jax.experimental.pallas import tpu as pltpu
from jax.experimental.pallas import tpu_sc as plsc

N = 10000
D = 128
E = 320000

NW = 32
K = 128
NCH = 80
EPW = NCH * K
E_PAD = NW * EPW
N_PAD = 10240
RPT = N_PAD // 16
ZR = 64

_mesh = plsc.VectorSubcoreMesh(core_axis_name="c", subcore_axis_name="s")


def _sc_scatter(F):
  @functools.partial(
      pl.kernel,
      out_type=jax.ShapeDtypeStruct((2, N_PAD, F), jnp.float32),
      mesh=_mesh,
      compiler_params=pltpu.CompilerParams(use_tc_tiling_on_sc=False),
      scratch_types=[
          pltpu.VMEM((NCH + 1, K), jnp.int32),
          pltpu.VMEM((NCH, K), jnp.int32),
          pltpu.VMEM((2, K, F), jnp.float32),
          pltpu.VMEM((ZR, F), jnp.float32),
          pltpu.VMEM_SHARED((N_PAD, F), jnp.float32),
          pltpu.SemaphoreType.DMA,
          pltpu.SemaphoreType.DMA,
      ],
  )
  def k(u_hbm, srcp_hbm, dstp_hbm, out_hbm, src_t, dst_t, rows, zbuf, acc,
        gsem, ssem):
    c = lax.axis_index("c")
    s = lax.axis_index("s")
    wid = s * 2 + c

    def zrow(r, carry):
      for t in range(F // 16):
        zbuf[r, pl.ds(t * 16, 16)] = jnp.zeros((16,), jnp.float32)
      return carry

    lax.fori_loop(0, ZR, zrow, 0)

    def zcp(i, carry):
      pltpu.sync_copy(zbuf, acc.at[pl.ds(s * RPT + i * ZR, ZR)])
      return carry

    lax.fori_loop(0, RPT // ZR, zcp, 0)

    pltpu.sync_copy(srcp_hbm.at[wid], src_t)
    pltpu.sync_copy(dstp_hbm.at[wid], dst_t)
    plsc.subcore_barrier()

    pltpu.async_copy(u_hbm.at[src_t.at[0]], rows.at[0], gsem)

    def gwait(b):
      pltpu.make_async_copy(u_hbm.at[src_t.at[0]], rows.at[b], gsem).wait()

    def swait(b):
      pltpu.make_async_copy(rows.at[b], acc.at[dst_t.at[0]], ssem).wait()

    def body(i, carry):
      j0 = 2 * i
      gwait(0)
      pltpu.async_copy(u_hbm.at[src_t.at[j0 + 1]], rows.at[1], gsem)
      pltpu.async_copy(rows.at[0], acc.at[dst_t.at[j0]], ssem, add=True)
      gwait(1)
      swait(0)
      pltpu.async_copy(u_hbm.at[src_t.at[j0 + 2]], rows.at[0], gsem)
      pltpu.async_copy(rows.at[1], acc.at[dst_t.at[j0 + 1]], ssem, add=True)
      swait(1)
      return carry

    lax.fori_loop(0, NCH // 2, body, 0)
    gwait(0)
    plsc.subcore_barrier()
    pltpu.sync_copy(acc.at[pl.ds(s * RPT, RPT)],
                    out_hbm.at[c, pl.ds(s * RPT, RPT)])

  return k


def _sc_degree():
  @functools.partial(
      pl.kernel,
      out_type=jax.ShapeDtypeStruct((2, N_PAD, 16), jnp.float32),
      mesh=_mesh,
      compiler_params=pltpu.CompilerParams(use_tc_tiling_on_sc=False),
      scratch_types=[
          pltpu.VMEM((NCH, K), jnp.int32),
          pltpu.VMEM((K, 16), jnp.float32),
          pltpu.VMEM((ZR, 16), jnp.float32),
          pltpu.VMEM_SHARED((N_PAD, 16), jnp.float32),
          pltpu.SemaphoreType.DMA,
      ],
  )
  def k(dstp_hbm, out_hbm, dst_t, ones_b, zbuf, acc, ssem):
    c = lax.axis_index("c")
    s = lax.axis_index("s")
    wid = s * 2 + c

    def zrow(r, carry):
      zbuf[r, pl.ds(0, 16)] = jnp.zeros((16,), jnp.float32)
      return carry

    lax.fori_loop(0, ZR, zrow, 0)

    def orow(r, carry):
      ones_b[r, pl.ds(0, 16)] = jnp.ones((16,), jnp.float32)
      return carry

    lax.fori_loop(0, K, orow, 0)

    def zcp(i, carry):
      pltpu.sync_copy(zbuf, acc.at[pl.ds(s * RPT + i * ZR, ZR)])
      return carry

    lax.fori_loop(0, RPT // ZR, zcp, 0)

    pltpu.sync_copy(dstp_hbm.at[wid], dst_t)
    plsc.subcore_barrier()

    def body(i, carry):
      for b in range(8):
        pltpu.async_copy(ones_b, acc.at[dst_t.at[i * 8 + b]], ssem, add=True)
      for b in range(8):
        pltpu.make_async_copy(ones_b, acc.at[dst_t.at[0]], ssem).wait()
      return carry

    lax.fori_loop(0, NCH // 8, body, 0)
    plsc.subcore_barrier()
    pltpu.sync_copy(acc.at[pl.ds(s * RPT, RPT)],
                    out_hbm.at[c, pl.ds(s * RPT, RPT)])

  return k


_B = 2000


def _deg_dinv(dpA, dpB):
  deg = dpA[0][:, :1] + dpB[0][:, :1] + 1.0
  return 1.0 / jnp.sqrt(deg)


def _tc1_body(x_ref, dpA, dpB, W_ref, o_ref):
  dinv = _deg_dinv(dpA, dpB)
  o_ref[...] = dinv * jnp.dot(x_ref[...], W_ref[...],
                              preferred_element_type=jnp.float32)


def _tc_mid_body(spA, spB, u_ref, dpA, dpB, W_ref, b_ref, o_ref):
  dinv = _deg_dinv(dpA, dpB)
  h = jnp.maximum(dinv * (spA[0] + spB[0] + u_ref[...]) + b_ref[:1], 0.0)
  o_ref[...] = dinv * jnp.dot(h, W_ref[...], preferred_element_type=jnp.float32)


def _tc_out_body(spA, spB, u_ref, dpA, dpB, b_ref, Wl_ref, bl_ref, o_ref):
  dinv = _deg_dinv(dpA, dpB)
  h = dinv * (spA[0] + spB[0] + u_ref[...]) + b_ref[:1]
  o_ref[...] = jnp.dot(h, Wl_ref[...],
                       preferred_element_type=jnp.float32) + bl_ref[:1]


def _row_spec(Fdim):
  return pl.BlockSpec((_B, Fdim), lambda i: (i, 0))


def _part_spec(Fdim):
  n = Fdim

  def a(i):
    return (0, i, 0)

  def b(i):
    return (1, i, 0)

  return (pl.BlockSpec((1, _B, n), a), pl.BlockSpec((1, _B, n), b))


def _full_spec(shape):
  nd = len(shape)
  return pl.BlockSpec(shape, lambda i: (0,) * nd)


def kernel(x, edge_index, W1, b1, W2, b2, W3, b3, Wl, bl):
  src = edge_index[0].astype(jnp.int32)
  dst = edge_index[1].astype(jnp.int32)

  pad = E_PAD - E
  ar = jnp.arange(pad, dtype=jnp.int32)
  srcp = jnp.concatenate([src, ar % N]).reshape(NW, NCH, K)
  extra = (jnp.arange(NW * K, dtype=jnp.int32) % N).reshape(NW, 1, K)
  src3 = jnp.concatenate([srcp, extra], axis=1)
  dst3 = jnp.concatenate([dst, N + ar % (N_PAD - N)]).reshape(NW, NCH, K)

  degp = _sc_degree()(dst3)

  grid = (N // _B,)
  dspecs = _part_spec(16)

  b1r = jnp.broadcast_to(b1[None, :], (8, b1.shape[0]))
  b2r = jnp.broadcast_to(b2[None, :], (8, b2.shape[0]))
  b3r = jnp.broadcast_to(b3[None, :], (8, b3.shape[0]))
  blr = jnp.broadcast_to(bl[None, :], (8, bl.shape[0]))

  u1 = pl.pallas_call(
      _tc1_body,
      grid=grid,
      in_specs=[_row_spec(D), *dspecs, _full_spec(W1.shape)],
      out_specs=_row_spec(64),
      out_shape=jax.ShapeDtypeStruct((N, 64), jnp.float32),
  )(x, degp, degp, W1)

  s1 = _sc_scatter(64)(u1, src3, dst3)

  u2 = pl.pallas_call(
      _tc_mid_body,
      grid=grid,
      in_specs=[*_part_spec(64), _row_spec(64), *dspecs,
                _full_spec(W2.shape), _full_spec((8, 64))],
      out_specs=_row_spec(32),
      out_shape=jax.ShapeDtypeStruct((N, 32), jnp.float32),
  )(s1, s1, u1, degp, degp, W2, b1r)

  s2 = _sc_scatter(32)(u2, src3, dst3)

  u3 = pl.pallas_call(
      _tc_mid_body,
      grid=grid,
      in_specs=[*_part_spec(32), _row_spec(32), *dspecs,
                _full_spec(W3.shape), _full_spec((8, 32))],
      out_specs=_row_spec(16),
      out_shape=jax.ShapeDtypeStruct((N, 16), jnp.float32),
  )(s2, s2, u2, degp, degp, W3, b2r)

  s3 = _sc_scatter(16)(u3, src3, dst3)

  out = pl.pallas_call(
      _tc_out_body,
      grid=grid,
      in_specs=[*_part_spec(16), _row_spec(16), *dspecs,
                _full_spec((8, 16)), _full_spec(Wl.shape), _full_spec((8, 7))],
      out_specs=_row_spec(7),
      out_shape=jax.ShapeDtypeStruct((N, 7), jnp.float32),
  )(s3, s3, u3, degp, degp, b3r, Wl, blr)

  return out


# speedup vs baseline: 40.7872x; 1.2760x over previous
import functools

import jax
import jax.numpy as jnp
from jax import lax
from jax.experimental import pallas as pl
from jax.experimental.pallas import tpu as pltpu
from jax.experimental.pallas import tpu_sc as plsc

N = 10000
D = 128
E = 320000

NW = 32
K = 128
NCH = 80
EPW = NCH * K
E_PAD = NW * EPW
N_PAD = 10240
RPT = N_PAD // 16
ZR = 64

_mesh = plsc.VectorSubcoreMesh(core_axis_name="c", subcore_axis_name="s")


def _sc_scatter(F):
  @functools.partial(
      pl.kernel,
      out_type=jax.ShapeDtypeStruct((2, N_PAD, F), jnp.float32),
      mesh=_mesh,
      compiler_params=pltpu.CompilerParams(use_tc_tiling_on_sc=False),
      scratch_types=[
          pltpu.VMEM((NCH + 2, K), jnp.int32),
          pltpu.VMEM((NCH + 1, K), jnp.int32),
          pltpu.VMEM((4, K, F), jnp.float32),
          pltpu.VMEM((ZR, F), jnp.float32),
          pltpu.VMEM_SHARED((N_PAD, F), jnp.float32),
          pltpu.SemaphoreType.DMA,
          pltpu.SemaphoreType.DMA,
          pltpu.SemaphoreType.DMA,
          pltpu.SemaphoreType.DMA,
          pltpu.SemaphoreType.DMA,
          pltpu.SemaphoreType.DMA,
          pltpu.SemaphoreType.DMA,
          pltpu.SemaphoreType.DMA,
      ],
  )
  def k(u_hbm, srcp_hbm, dstp_hbm, out_hbm, src_t, dst_t, rows, zbuf, acc,
        g0, g1, g2, g3, s0, s1, s2, s3):
    gsems = (g0, g1, g2, g3)
    ssems = (s0, s1, s2, s3)
    c = lax.axis_index("c")
    s = lax.axis_index("s")
    wid = s * 2 + c

    def zrow(r, carry):
      for t in range(F // 16):
        zbuf[r, pl.ds(t * 16, 16)] = jnp.zeros((16,), jnp.float32)
      return carry

    lax.fori_loop(0, ZR, zrow, 0)

    def zcp(i, carry):
      pltpu.sync_copy(zbuf, acc.at[pl.ds(s * RPT + i * ZR, ZR)])
      return carry

    lax.fori_loop(0, RPT // ZR, zcp, 0)

    pltpu.sync_copy(srcp_hbm.at[wid], src_t)
    pltpu.sync_copy(dstp_hbm.at[wid], dst_t)
    plsc.subcore_barrier()

    def gfire(j, b):
      pltpu.async_copy(u_hbm.at[src_t.at[j]], rows.at[b], gsems[b])

    def gwait(b):
      pltpu.make_async_copy(u_hbm.at[src_t.at[0]], rows.at[b],
                            gsems[b]).wait()

    def sfire(j, b):
      pltpu.async_copy(rows.at[b], acc.at[dst_t.at[j]], ssems[b], add=True)

    def swait(b):
      pltpu.make_async_copy(rows.at[b], acc.at[dst_t.at[0]], ssems[b]).wait()

    gfire(0, 0)
    gfire(1, 1)
    sfire(NCH, 2)
    sfire(NCH, 3)

    def body(i, carry):
      j0 = 4 * i
      for b in range(4):
        j = j0 + b
        gwait(b)
        sfire(j, b)
        swait((b + 2) % 4)
        gfire(j + 2, (b + 2) % 4)
      return carry

    lax.fori_loop(0, NCH // 4, body, 0)
    swait(2)
    swait(3)
    gwait(0)
    gwait(1)
    plsc.subcore_barrier()
    pltpu.sync_copy(acc.at[pl.ds(s * RPT, RPT)],
                    out_hbm.at[c, pl.ds(s * RPT, RPT)])

  return k


def _sc_degree():
  @functools.partial(
      pl.kernel,
      out_type=jax.ShapeDtypeStruct((2, N_PAD, 16), jnp.float32),
      mesh=_mesh,
      compiler_params=pltpu.CompilerParams(use_tc_tiling_on_sc=False),
      scratch_types=[
          pltpu.VMEM((NCH + 1, K), jnp.int32),
          pltpu.VMEM((K, 16), jnp.float32),
          pltpu.VMEM((ZR, 16), jnp.float32),
          pltpu.VMEM_SHARED((N_PAD, 16), jnp.float32),
          pltpu.SemaphoreType.DMA,
      ],
  )
  def k(dstp_hbm, out_hbm, dst_t, ones_b, zbuf, acc, ssem):
    c = lax.axis_index("c")
    s = lax.axis_index("s")
    wid = s * 2 + c

    def zrow(r, carry):
      zbuf[r, pl.ds(0, 16)] = jnp.zeros((16,), jnp.float32)
      return carry

    lax.fori_loop(0, ZR, zrow, 0)

    def orow(r, carry):
      ones_b[r, pl.ds(0, 16)] = jnp.ones((16,), jnp.float32)
      return carry

    lax.fori_loop(0, K, orow, 0)

    def zcp(i, carry):
      pltpu.sync_copy(zbuf, acc.at[pl.ds(s * RPT + i * ZR, ZR)])
      return carry

    lax.fori_loop(0, RPT // ZR, zcp, 0)

    pltpu.sync_copy(dstp_hbm.at[wid], dst_t)
    plsc.subcore_barrier()

    def body(i, carry):
      for b in range(8):
        pltpu.async_copy(ones_b, acc.at[dst_t.at[i * 8 + b]], ssem, add=True)
      for b in range(8):
        pltpu.make_async_copy(ones_b, acc.at[dst_t.at[0]], ssem).wait()
      return carry

    lax.fori_loop(0, NCH // 8, body, 0)
    plsc.subcore_barrier()
    pltpu.sync_copy(acc.at[pl.ds(s * RPT, RPT)],
                    out_hbm.at[c, pl.ds(s * RPT, RPT)])

  return k


_B = 2000


def _deg_dinv(dpA, dpB):
  deg = dpA[0][:, :1] + dpB[0][:, :1] + 1.0
  return 1.0 / jnp.sqrt(deg)


def _tc1_body(x_ref, dpA, dpB, W_ref, o_ref):
  dinv = _deg_dinv(dpA, dpB)
  o_ref[...] = dinv * jnp.dot(x_ref[...], W_ref[...],
                              preferred_element_type=jnp.float32)


def _tc_mid_body(spA, spB, u_ref, dpA, dpB, W_ref, b_ref, o_ref):
  dinv = _deg_dinv(dpA, dpB)
  h = jnp.maximum(dinv * (spA[0] + spB[0] + u_ref[...]) + b_ref[:1], 0.0)
  o_ref[...] = dinv * jnp.dot(h, W_ref[...], preferred_element_type=jnp.float32)


def _tc_out_body(spA, spB, u_ref, dpA, dpB, b_ref, Wl_ref, bl_ref, o_ref):
  dinv = _deg_dinv(dpA, dpB)
  h = dinv * (spA[0] + spB[0] + u_ref[...]) + b_ref[:1]
  o_ref[...] = jnp.dot(h, Wl_ref[...],
                       preferred_element_type=jnp.float32) + bl_ref[:1]


def _row_spec(Fdim):
  return pl.BlockSpec((_B, Fdim), lambda i: (i, 0))


def _part_spec(Fdim):
  n = Fdim

  def a(i):
    return (0, i, 0)

  def b(i):
    return (1, i, 0)

  return (pl.BlockSpec((1, _B, n), a), pl.BlockSpec((1, _B, n), b))


def _full_spec(shape):
  nd = len(shape)
  return pl.BlockSpec(shape, lambda i: (0,) * nd)


def kernel(x, edge_index, W1, b1, W2, b2, W3, b3, Wl, bl):
  src = edge_index[0].astype(jnp.int32)
  dst = edge_index[1].astype(jnp.int32)

  pad = E_PAD - E
  ar = jnp.arange(pad, dtype=jnp.int32)
  srcp = jnp.concatenate([src, ar % N]).reshape(NW, NCH, K)
  extra = (jnp.arange(NW * 2 * K, dtype=jnp.int32) % N).reshape(NW, 2, K)
  src3 = jnp.concatenate([srcp, extra], axis=1)
  dstp = jnp.concatenate([dst, N + ar % (N_PAD - N)]).reshape(NW, NCH, K)
  dead = (N + jnp.arange(NW * K, dtype=jnp.int32) % (N_PAD - N)).reshape(
      NW, 1, K)
  dst3 = jnp.concatenate([dstp, dead], axis=1)

  degp = _sc_degree()(dst3)

  grid = (N // _B,)
  dspecs = _part_spec(16)

  b1r = jnp.broadcast_to(b1[None, :], (8, b1.shape[0]))
  b2r = jnp.broadcast_to(b2[None, :], (8, b2.shape[0]))
  b3r = jnp.broadcast_to(b3[None, :], (8, b3.shape[0]))
  blr = jnp.broadcast_to(bl[None, :], (8, bl.shape[0]))

  u1 = pl.pallas_call(
      _tc1_body,
      grid=grid,
      in_specs=[_row_spec(D), *dspecs, _full_spec(W1.shape)],
      out_specs=_row_spec(64),
      out_shape=jax.ShapeDtypeStruct((N, 64), jnp.float32),
  )(x, degp, degp, W1)

  s1 = _sc_scatter(64)(u1, src3, dst3)

  u2 = pl.pallas_call(
      _tc_mid_body,
      grid=grid,
      in_specs=[*_part_spec(64), _row_spec(64), *dspecs,
                _full_spec(W2.shape), _full_spec((8, 64))],
      out_specs=_row_spec(32),
      out_shape=jax.ShapeDtypeStruct((N, 32), jnp.float32),
  )(s1, s1, u1, degp, degp, W2, b1r)

  s2 = _sc_scatter(32)(u2, src3, dst3)

  u3 = pl.pallas_call(
      _tc_mid_body,
      grid=grid,
      in_specs=[*_part_spec(32), _row_spec(32), *dspecs,
                _full_spec(W3.shape), _full_spec((8, 32))],
      out_specs=_row_spec(16),
      out_shape=jax.ShapeDtypeStruct((N, 16), jnp.float32),
  )(s2, s2, u2, degp, degp, W3, b2r)

  s3 = _sc_scatter(16)(u3, src3, dst3)

  out = pl.pallas_call(
      _tc_out_body,
      grid=grid,
      in_specs=[*_part_spec(16), _row_spec(16), *dspecs,
                _full_spec((8, 16)), _full_spec(Wl.shape), _full_spec((8, 7))],
      out_specs=_row_spec(7),
      out_shape=jax.ShapeDtypeStruct((N, 7), jnp.float32),
  )(s3, s3, u3, degp, degp, b3r, Wl, blr)

  return out


# speedup vs baseline: 43.4107x; 1.0643x over previous
import functools

import jax
import jax.numpy as jnp
from jax import lax
from jax.experimental import pallas as pl
from jax.experimental.pallas import tpu as pltpu
from jax.experimental.pallas import tpu_sc as plsc

N = 10000
D = 128
E = 320000

NW = 32
K = 256
NCH = 42
EPW = NCH * K
E_PAD = NW * EPW
N_PAD = 10240
RPT = N_PAD // 16
ZR = 64

_mesh = plsc.VectorSubcoreMesh(core_axis_name="c", subcore_axis_name="s")


def _sc_scatter(F):
  @functools.partial(
      pl.kernel,
      out_type=jax.ShapeDtypeStruct((2, N_PAD, F), jnp.float32),
      mesh=_mesh,
      compiler_params=pltpu.CompilerParams(use_tc_tiling_on_sc=False),
      scratch_types=[
          pltpu.VMEM((NCH + 2, K), jnp.int32),
          pltpu.VMEM((NCH + 1, K), jnp.int32),
          pltpu.VMEM((3, K, F), jnp.float32),
          pltpu.VMEM((ZR, F), jnp.float32),
          pltpu.VMEM_SHARED((N_PAD, F), jnp.float32),
          pltpu.SemaphoreType.DMA,
          pltpu.SemaphoreType.DMA,
          pltpu.SemaphoreType.DMA,
          pltpu.SemaphoreType.DMA,
          pltpu.SemaphoreType.DMA,
          pltpu.SemaphoreType.DMA,
      ],
  )
  def k(u_hbm, srcp_hbm, dstp_hbm, out_hbm, src_t, dst_t, rows, zbuf, acc,
        g0, g1, g2, s0, s1, s2):
    gsems = (g0, g1, g2)
    ssems = (s0, s1, s2)
    c = lax.axis_index("c")
    s = lax.axis_index("s")
    wid = s * 2 + c

    def zrow(r, carry):
      for t in range(F // 16):
        zbuf[r, pl.ds(t * 16, 16)] = jnp.zeros((16,), jnp.float32)
      return carry

    lax.fori_loop(0, ZR, zrow, 0)

    def zcp(i, carry):
      pltpu.sync_copy(zbuf, acc.at[pl.ds(s * RPT + i * ZR, ZR)])
      return carry

    lax.fori_loop(0, RPT // ZR, zcp, 0)

    pltpu.sync_copy(srcp_hbm.at[wid], src_t)
    pltpu.sync_copy(dstp_hbm.at[wid], dst_t)
    plsc.subcore_barrier()

    def gfire(j, b):
      pltpu.async_copy(u_hbm.at[src_t.at[j]], rows.at[b], gsems[b])

    def gwait(b):
      pltpu.make_async_copy(u_hbm.at[src_t.at[0]], rows.at[b],
                            gsems[b]).wait()

    def sfire(j, b):
      pltpu.async_copy(rows.at[b], acc.at[dst_t.at[j]], ssems[b], add=True)

    def swait(b):
      pltpu.make_async_copy(rows.at[b], acc.at[dst_t.at[0]], ssems[b]).wait()

    gfire(0, 0)
    gfire(1, 1)
    sfire(NCH, 2)

    def body(i, carry):
      j0 = 6 * i
      for t in range(6):
        j = j0 + t
        b = t % 3
        gwait(b)
        sfire(j, b)
        swait((b + 2) % 3)
        gfire(j + 2, (b + 2) % 3)
      return carry

    lax.fori_loop(0, NCH // 6, body, 0)
    swait(2)
    gwait(0)
    gwait(1)
    plsc.subcore_barrier()
    pltpu.sync_copy(acc.at[pl.ds(s * RPT, RPT)],
                    out_hbm.at[c, pl.ds(s * RPT, RPT)])

  return k


def _sc_degree():
  @functools.partial(
      pl.kernel,
      out_type=jax.ShapeDtypeStruct((2, N_PAD, 16), jnp.float32),
      mesh=_mesh,
      compiler_params=pltpu.CompilerParams(use_tc_tiling_on_sc=False),
      scratch_types=[
          pltpu.VMEM((NCH + 1, K), jnp.int32),
          pltpu.VMEM((K, 16), jnp.float32),
          pltpu.VMEM((ZR, 16), jnp.float32),
          pltpu.VMEM_SHARED((N_PAD, 16), jnp.float32),
          pltpu.SemaphoreType.DMA,
      ],
  )
  def k(dstp_hbm, out_hbm, dst_t, ones_b, zbuf, acc, ssem):
    c = lax.axis_index("c")
    s = lax.axis_index("s")
    wid = s * 2 + c

    def zrow(r, carry):
      zbuf[r, pl.ds(0, 16)] = jnp.zeros((16,), jnp.float32)
      return carry

    lax.fori_loop(0, ZR, zrow, 0)

    def orow(r, carry):
      ones_b[r, pl.ds(0, 16)] = jnp.ones((16,), jnp.float32)
      return carry

    lax.fori_loop(0, K, orow, 0)

    def zcp(i, carry):
      pltpu.sync_copy(zbuf, acc.at[pl.ds(s * RPT + i * ZR, ZR)])
      return carry

    lax.fori_loop(0, RPT // ZR, zcp, 0)

    pltpu.sync_copy(dstp_hbm.at[wid], dst_t)
    plsc.subcore_barrier()

    def body(i, carry):
      for b in range(6):
        pltpu.async_copy(ones_b, acc.at[dst_t.at[i * 6 + b]], ssem, add=True)
      for b in range(6):
        pltpu.make_async_copy(ones_b, acc.at[dst_t.at[0]], ssem).wait()
      return carry

    lax.fori_loop(0, NCH // 6, body, 0)
    plsc.subcore_barrier()
    pltpu.sync_copy(acc.at[pl.ds(s * RPT, RPT)],
                    out_hbm.at[c, pl.ds(s * RPT, RPT)])

  return k


_B = 10000


def _deg_dinv(dpA, dpB):
  deg = dpA[0][:, :1] + dpB[0][:, :1] + 1.0
  return 1.0 / jnp.sqrt(deg)


def _tc1_body(x_ref, dpA, dpB, W_ref, o_ref):
  dinv = _deg_dinv(dpA, dpB)
  o_ref[...] = dinv * jnp.dot(x_ref[...], W_ref[...],
                              preferred_element_type=jnp.float32)


def _tc_mid_body(spA, spB, u_ref, dpA, dpB, W_ref, b_ref, o_ref):
  dinv = _deg_dinv(dpA, dpB)
  h = jnp.maximum(dinv * (spA[0] + spB[0] + u_ref[...]) + b_ref[:1], 0.0)
  o_ref[...] = dinv * jnp.dot(h, W_ref[...], preferred_element_type=jnp.float32)


def _tc_out_body(spA, spB, u_ref, dpA, dpB, b_ref, Wl_ref, bl_ref, o_ref):
  dinv = _deg_dinv(dpA, dpB)
  h = dinv * (spA[0] + spB[0] + u_ref[...]) + b_ref[:1]
  o_ref[...] = jnp.dot(h, Wl_ref[...],
                       preferred_element_type=jnp.float32) + bl_ref[:1]


def _row_spec(Fdim):
  return pl.BlockSpec((_B, Fdim), lambda i: (i, 0))


def _part_spec(Fdim):
  n = Fdim

  def a(i):
    return (0, i, 0)

  def b(i):
    return (1, i, 0)

  return (pl.BlockSpec((1, _B, n), a), pl.BlockSpec((1, _B, n), b))


def _full_spec(shape):
  nd = len(shape)
  return pl.BlockSpec(shape, lambda i: (0,) * nd)


def kernel(x, edge_index, W1, b1, W2, b2, W3, b3, Wl, bl):
  src = edge_index[0].astype(jnp.int32)
  dst = edge_index[1].astype(jnp.int32)

  pad = E_PAD - E
  ar = jnp.arange(pad, dtype=jnp.int32)
  srcp = jnp.concatenate([src, ar % N]).reshape(NW, NCH, K)
  extra = (jnp.arange(NW * 2 * K, dtype=jnp.int32) % N).reshape(NW, 2, K)
  src3 = jnp.concatenate([srcp, extra], axis=1)
  dstp = jnp.concatenate([dst, N + ar % (N_PAD - N)]).reshape(NW, NCH, K)
  dead = (N + jnp.arange(NW * K, dtype=jnp.int32) % (N_PAD - N)).reshape(
      NW, 1, K)
  dst3 = jnp.concatenate([dstp, dead], axis=1)

  degp = _sc_degree()(dst3)

  grid = (N // _B,)
  dspecs = _part_spec(16)

  b1r = jnp.broadcast_to(b1[None, :], (8, b1.shape[0]))
  b2r = jnp.broadcast_to(b2[None, :], (8, b2.shape[0]))
  b3r = jnp.broadcast_to(b3[None, :], (8, b3.shape[0]))
  blr = jnp.broadcast_to(bl[None, :], (8, bl.shape[0]))

  u1 = pl.pallas_call(
      _tc1_body,
      grid=grid,
      in_specs=[_row_spec(D), *dspecs, _full_spec(W1.shape)],
      out_specs=_row_spec(64),
      out_shape=jax.ShapeDtypeStruct((N, 64), jnp.float32),
  )(x, degp, degp, W1)

  s1 = _sc_scatter(64)(u1, src3, dst3)

  u2 = pl.pallas_call(
      _tc_mid_body,
      grid=grid,
      in_specs=[*_part_spec(64), _row_spec(64), *dspecs,
                _full_spec(W2.shape), _full_spec((8, 64))],
      out_specs=_row_spec(32),
      out_shape=jax.ShapeDtypeStruct((N, 32), jnp.float32),
  )(s1, s1, u1, degp, degp, W2, b1r)

  s2 = _sc_scatter(32)(u2, src3, dst3)

  u3 = pl.pallas_call(
      _tc_mid_body,
      grid=grid,
      in_specs=[*_part_spec(32), _row_spec(32), *dspecs,
                _full_spec(W3.shape), _full_spec((8, 32))],
      out_specs=_row_spec(16),
      out_shape=jax.ShapeDtypeStruct((N, 16), jnp.float32),
  )(s2, s2, u2, degp, degp, W3, b2r)

  s3 = _sc_scatter(16)(u3, src3, dst3)

  out = pl.pallas_call(
      _tc_out_body,
      grid=grid,
      in_specs=[*_part_spec(16), _row_spec(16), *dspecs,
                _full_spec((8, 16)), _full_spec(Wl.shape), _full_spec((8, 7))],
      out_specs=_row_spec(7),
      out_shape=jax.ShapeDtypeStruct((N, 7), jnp.float32),
  )(s3, s3, u3, degp, degp, b3r, Wl, blr)

  return out


# speedup vs baseline: 45.1605x; 1.0403x over previous
import functools

import jax
import jax.numpy as jnp
from jax import lax
from jax.experimental import pallas as pl
from jax.experimental.pallas import tpu as pltpu
from jax.experimental.pallas import tpu_sc as plsc

N = 10000
D = 128
E = 320000

NW = 32
K = 256
NCH = 40
EPW = NCH * K
E_PAD = NW * EPW
N_PAD = 10240
RPT = N_PAD // 16
ZR = 64

_mesh = plsc.VectorSubcoreMesh(core_axis_name="c", subcore_axis_name="s")


def _sc_scatter(F):
  @functools.partial(
      pl.kernel,
      out_type=jax.ShapeDtypeStruct((2, N_PAD, F), jnp.float32),
      mesh=_mesh,
      compiler_params=pltpu.CompilerParams(use_tc_tiling_on_sc=False),
      scratch_types=[
          pltpu.VMEM((NCH + 2, K), jnp.int32),
          pltpu.VMEM((NCH + 1, K), jnp.int32),
          pltpu.VMEM((3, K, F), jnp.float32),
          pltpu.VMEM((ZR, F), jnp.float32),
          pltpu.VMEM_SHARED((N_PAD, F), jnp.float32),
          pltpu.SemaphoreType.DMA,
          pltpu.SemaphoreType.DMA,
          pltpu.SemaphoreType.DMA,
          pltpu.SemaphoreType.DMA,
          pltpu.SemaphoreType.DMA,
          pltpu.SemaphoreType.DMA,
      ],
  )
  def k(u_hbm, srcp_hbm, dstp_hbm, out_hbm, src_t, dst_t, rows, zbuf, acc,
        g0, g1, g2, s0, s1, s2):
    gsems = (g0, g1, g2)
    ssems = (s0, s1, s2)
    c = lax.axis_index("c")
    s = lax.axis_index("s")
    wid = s * 2 + c

    def zrow(r, carry):
      for t in range(F // 16):
        zbuf[r, pl.ds(t * 16, 16)] = jnp.zeros((16,), jnp.float32)
      return carry

    lax.fori_loop(0, ZR, zrow, 0)

    def zcp(i, carry):
      pltpu.sync_copy(zbuf, acc.at[pl.ds(s * RPT + i * ZR, ZR)])
      return carry

    lax.fori_loop(0, RPT // ZR, zcp, 0)

    pltpu.sync_copy(srcp_hbm.at[wid], src_t)
    pltpu.sync_copy(dstp_hbm.at[wid], dst_t)
    plsc.subcore_barrier()

    def gfire(j, b):
      pltpu.async_copy(u_hbm.at[src_t.at[j]], rows.at[b], gsems[b])

    def gwait(b):
      pltpu.make_async_copy(u_hbm.at[src_t.at[0]], rows.at[b],
                            gsems[b]).wait()

    def sfire(j, b):
      pltpu.async_copy(rows.at[b], acc.at[dst_t.at[j]], ssems[b], add=True)

    def swait(b):
      pltpu.make_async_copy(rows.at[b], acc.at[dst_t.at[0]], ssems[b]).wait()

    gfire(0, 0)
    gfire(1, 1)
    sfire(NCH, 2)

    def body(i, carry):
      j0 = 3 * i
      for t in range(3):
        j = j0 + t
        gwait(t)
        sfire(j, t)
        swait((t + 2) % 3)
        gfire(j + 2, (t + 2) % 3)
      return carry

    lax.fori_loop(0, (NCH - 1) // 3, body, 0)
    gwait(0)
    sfire(NCH - 1, 0)
    swait(2)
    gfire(NCH + 1, 2)
    swait(0)
    gwait(1)
    gwait(2)
    plsc.subcore_barrier()
    pltpu.sync_copy(acc.at[pl.ds(s * RPT, RPT)],
                    out_hbm.at[c, pl.ds(s * RPT, RPT)])

  return k


def _sc_degree():
  @functools.partial(
      pl.kernel,
      out_type=jax.ShapeDtypeStruct((2, N_PAD, 16), jnp.float32),
      mesh=_mesh,
      compiler_params=pltpu.CompilerParams(use_tc_tiling_on_sc=False),
      scratch_types=[
          pltpu.VMEM((NCH + 1, K), jnp.int32),
          pltpu.VMEM((K, 16), jnp.float32),
          pltpu.VMEM((ZR, 16), jnp.float32),
          pltpu.VMEM_SHARED((N_PAD, 16), jnp.float32),
          pltpu.SemaphoreType.DMA,
      ],
  )
  def k(dstp_hbm, out_hbm, dst_t, ones_b, zbuf, acc, ssem):
    c = lax.axis_index("c")
    s = lax.axis_index("s")
    wid = s * 2 + c

    def zrow(r, carry):
      zbuf[r, pl.ds(0, 16)] = jnp.zeros((16,), jnp.float32)
      return carry

    lax.fori_loop(0, ZR, zrow, 0)

    def orow(r, carry):
      ones_b[r, pl.ds(0, 16)] = jnp.ones((16,), jnp.float32)
      return carry

    lax.fori_loop(0, K, orow, 0)

    def zcp(i, carry):
      pltpu.sync_copy(zbuf, acc.at[pl.ds(s * RPT + i * ZR, ZR)])
      return carry

    lax.fori_loop(0, RPT // ZR, zcp, 0)

    pltpu.sync_copy(dstp_hbm.at[wid], dst_t)
    plsc.subcore_barrier()

    def body(i, carry):
      for b in range(8):
        pltpu.async_copy(ones_b, acc.at[dst_t.at[i * 8 + b]], ssem, add=True)
      for b in range(8):
        pltpu.make_async_copy(ones_b, acc.at[dst_t.at[0]], ssem).wait()
      return carry

    lax.fori_loop(0, NCH // 8, body, 0)
    plsc.subcore_barrier()
    pltpu.sync_copy(acc.at[pl.ds(s * RPT, RPT)],
                    out_hbm.at[c, pl.ds(s * RPT, RPT)])

  return k


_B = 10000


def _tc_mm_body(x_ref, W_ref, o_ref):
  o_ref[...] = jnp.dot(x_ref[...], W_ref[...],
                       preferred_element_type=jnp.float32)


def _tc_scale_body(p_ref, dpA, dpB, ou_ref, od_ref):
  deg = dpA[0][:, :1] + dpB[0][:, :1] + 1.0
  dinv = 1.0 / jnp.sqrt(deg)
  ou_ref[...] = dinv * p_ref[...]
  od_ref[...] = jnp.broadcast_to(dinv, od_ref.shape)


def _tc_mid_body(spA, spB, u_ref, dv_ref, W_ref, b_ref, o_ref):
  dinv = dv_ref[:, :1]
  h = jnp.maximum(dinv * (spA[0] + spB[0] + u_ref[...]) + b_ref[:1], 0.0)
  o_ref[...] = dinv * jnp.dot(h, W_ref[...], preferred_element_type=jnp.float32)


def _tc_out_body(spA, spB, u_ref, dv_ref, b_ref, Wl_ref, bl_ref, o_ref):
  dinv = dv_ref[:, :1]
  h = dinv * (spA[0] + spB[0] + u_ref[...]) + b_ref[:1]
  o_ref[...] = jnp.dot(h, Wl_ref[...],
                       preferred_element_type=jnp.float32) + bl_ref[:1]


def _row_spec(Fdim):
  return pl.BlockSpec((_B, Fdim), lambda i: (i, 0))


def _part_spec(Fdim):
  n = Fdim

  def a(i):
    return (0, i, 0)

  def b(i):
    return (1, i, 0)

  return (pl.BlockSpec((1, _B, n), a), pl.BlockSpec((1, _B, n), b))


def _full_spec(shape):
  nd = len(shape)
  return pl.BlockSpec(shape, lambda i: (0,) * nd)


def kernel(x, edge_index, W1, b1, W2, b2, W3, b3, Wl, bl):
  src = edge_index[0].astype(jnp.int32)
  dst = edge_index[1].astype(jnp.int32)

  pad = E_PAD - E
  ar = jnp.arange(pad, dtype=jnp.int32)
  dstp = jnp.concatenate([dst, N + ar % (N_PAD - N)]).reshape(NW, NCH, K)
  dead = (N + jnp.arange(NW * K, dtype=jnp.int32) % (N_PAD - N)).reshape(
      NW, 1, K)
  dst3 = jnp.concatenate([dstp, dead], axis=1)
  dst3 = lax.optimization_barrier(dst3)

  degp = _sc_degree()(dst3)

  srcp = jnp.concatenate([src, ar % N]).reshape(NW, NCH, K)
  extra = (jnp.arange(NW * 2 * K, dtype=jnp.int32) % N).reshape(NW, 2, K)
  src3 = jnp.concatenate([srcp, extra], axis=1)

  grid = (N // _B,)

  b1r = jnp.broadcast_to(b1[None, :], (8, b1.shape[0]))
  b2r = jnp.broadcast_to(b2[None, :], (8, b2.shape[0]))
  b3r = jnp.broadcast_to(b3[None, :], (8, b3.shape[0]))
  blr = jnp.broadcast_to(bl[None, :], (8, bl.shape[0]))

  p1 = pl.pallas_call(
      _tc_mm_body,
      grid=grid,
      in_specs=[_row_spec(D), _full_spec(W1.shape)],
      out_specs=_row_spec(64),
      out_shape=jax.ShapeDtypeStruct((N, 64), jnp.float32),
  )(x, W1)

  u1, dv = pl.pallas_call(
      _tc_scale_body,
      grid=grid,
      in_specs=[_row_spec(64), *_part_spec(16)],
      out_specs=[_row_spec(64), _row_spec(16)],
      out_shape=[jax.ShapeDtypeStruct((N, 64), jnp.float32),
                 jax.ShapeDtypeStruct((N, 16), jnp.float32)],
  )(p1, degp, degp)

  s1 = _sc_scatter(64)(u1, src3, dst3)

  u2 = pl.pallas_call(
      _tc_mid_body,
      grid=grid,
      in_specs=[*_part_spec(64), _row_spec(64), _row_spec(16),
                _full_spec(W2.shape), _full_spec((8, 64))],
      out_specs=_row_spec(32),
      out_shape=jax.ShapeDtypeStruct((N, 32), jnp.float32),
  )(s1, s1, u1, dv, W2, b1r)

  s2 = _sc_scatter(32)(u2, src3, dst3)

  u3 = pl.pallas_call(
      _tc_mid_body,
      grid=grid,
      in_specs=[*_part_spec(32), _row_spec(32), _row_spec(16),
                _full_spec(W3.shape), _full_spec((8, 32))],
      out_specs=_row_spec(16),
      out_shape=jax.ShapeDtypeStruct((N, 16), jnp.float32),
  )(s2, s2, u2, dv, W3, b2r)

  s3 = _sc_scatter(16)(u3, src3, dst3)

  out = pl.pallas_call(
      _tc_out_body,
      grid=grid,
      in_specs=[*_part_spec(16), _row_spec(16), _row_spec(16),
                _full_spec((8, 16)), _full_spec(Wl.shape), _full_spec((8, 7))],
      out_specs=_row_spec(7),
      out_shape=jax.ShapeDtypeStruct((N, 7), jnp.float32),
  )(s3, s3, u3, dv, b3r, Wl, blr)

  return out
